# baseline (speedup 1.0000x reference)
import functools

import jax
import jax.numpy as jnp
from jax import lax
from jax.experimental import pallas as pl
from jax.experimental.pallas import tpu as pltpu
from jax.experimental.pallas import tpu_sc as plsc

N_BINS = 2048
HIDDEN = 64
BATCH = 16384
FIELDS = 100
TOTAL = BATCH * FIELDS

NC = 2
NS = 16
NW = NC * NS
PER_W = TOTAL // NW
CHUNK = 512
NCHUNK = PER_W // CHUNK
SUB = 128
G = CHUNK // SUB
LANES = 16

_mesh = plsc.VectorSubcoreMesh(core_axis_name="c", subcore_axis_name="s")


@functools.partial(
    pl.kernel,
    mesh=_mesh,
    out_type=jax.ShapeDtypeStruct((TOTAL, HIDDEN), jnp.float32),
    scratch_types=[
        pltpu.VMEM((N_BINS,), jnp.float32),
        pltpu.VMEM((CHUNK,), jnp.float32),
        pltpu.VMEM((G, SUB), jnp.int32),
        pltpu.VMEM((CHUNK, HIDDEN), jnp.float32),
        pltpu.SemaphoreType.DMA,
    ],
    compiler_params=pltpu.CompilerParams(
        needs_layout_passes=False, use_tc_tiling_on_sc=False),
)
def _emb_kernel(x_hbm, bounds_hbm, table_hbm, out_hbm,
                bounds_v, x_v, idx_v, rows_v, sem):
    wid = lax.axis_index("s") * NC + lax.axis_index("c")
    base = wid * PER_W
    pltpu.sync_copy(bounds_hbm, bounds_v)

    def chunk_body(ci, carry):
        off = base + ci * CHUNK
        pltpu.sync_copy(x_hbm.at[pl.ds(off, CHUNK)], x_v)
        for g in range(G):
            for v in range(SUB // LANES):
                o = g * SUB + v * LANES
                xv = x_v[pl.ds(o, LANES)]
                k = jnp.clip((xv * jnp.float32(2046.0)).astype(jnp.int32),
                             1, N_BINS - 3)
                b0 = plsc.load_gather(bounds_v, [k - 1])
                b1 = plsc.load_gather(bounds_v, [k])
                b2 = plsc.load_gather(bounds_v, [k + 1])
                one = jnp.int32(1)
                zero = jnp.int32(0)
                idx = ((k - 1)
                       + jnp.where(b0 < xv, one, zero)
                       + jnp.where(b1 < xv, one, zero)
                       + jnp.where(b2 < xv, one, zero))
                idx_v[g, pl.ds(v * LANES, LANES)] = idx
        copies = [
            pltpu.async_copy(table_hbm.at[idx_v.at[g]],
                             rows_v.at[pl.ds(g * SUB, SUB)], sem)
            for g in range(G)
        ]
        for c in copies:
            c.wait()
        pltpu.sync_copy(rows_v, out_hbm.at[pl.ds(off, CHUNK)])
        return carry

    lax.fori_loop(0, NCHUNK, chunk_body, 0)


def kernel(x, table):
    bounds = jnp.linspace(0.0, 1.0, N_BINS - 1, dtype=jnp.float32)
    bounds = jnp.concatenate([bounds, jnp.ones((1,), jnp.float32)])
    out = _emb_kernel(x.reshape(TOTAL), bounds, table)
    out = out.reshape(BATCH, FIELDS, HIDDEN)
    return jnp.where(out == out, out, jnp.float32(0.0))


# speedup vs baseline: 3.8348x; 3.8348x over previous
import functools

import jax
import jax.numpy as jnp
from jax import lax
from jax.experimental import pallas as pl
from jax.experimental.pallas import tpu as pltpu
from jax.experimental.pallas import tpu_sc as plsc

N_BINS = 2048
HIDDEN = 64
BATCH = 16384
FIELDS = 100

NC = 2
NS = 16
NW = NC * NS
LANES = 16

TPB = 128
NT0 = BATCH // TPB
T0_PER_W = NT0 // NW
H8 = HIDDEN // 8

_mesh = plsc.VectorSubcoreMesh(core_axis_name="c", subcore_axis_name="s")


@functools.partial(
    pl.kernel,
    mesh=_mesh,
    out_type=jax.ShapeDtypeStruct((FIELDS, H8, NT0, 8, TPB), jnp.float32),
    scratch_types=[
        pltpu.VMEM((N_BINS,), jnp.float32),
        pltpu.VMEM((FIELDS, TPB), jnp.float32),
        pltpu.VMEM((FIELDS, TPB), jnp.int32),
        pltpu.VMEM((2, TPB, HIDDEN), jnp.float32),
        pltpu.VMEM((2, H8, 1, 8, TPB), jnp.float32),
        pltpu.SemaphoreType.DMA,
        pltpu.SemaphoreType.DMA,
        pltpu.SemaphoreType.DMA,
        pltpu.SemaphoreType.DMA,
    ],
    compiler_params=pltpu.CompilerParams(
        needs_layout_passes=False, use_tc_tiling_on_sc=False),
)
def _emb_kernel(xT_hbm, bounds_hbm, table_hbm, out_hbm,
                bounds_v, x_v, idx_v, rows_v, tile_v,
                gsem0, gsem1, osem0, osem1):
    wid = lax.axis_index("s") * NC + lax.axis_index("c")
    pltpu.sync_copy(bounds_hbm, bounds_v)
    bidx = [lax.iota(jnp.int32, LANES) + LANES * L for L in range(TPB // LANES)]

    def compute_idx(f, carry):
        for v in range(TPB // LANES):
            xv = x_v[f, pl.ds(v * LANES, LANES)]
            k = jnp.clip((xv * jnp.float32(2046.0)).astype(jnp.int32),
                         1, N_BINS - 3)
            b0 = plsc.load_gather(bounds_v, [k - 1])
            b1 = plsc.load_gather(bounds_v, [k])
            b2 = plsc.load_gather(bounds_v, [k + 1])
            one = jnp.int32(1)
            zero = jnp.int32(0)
            idx = ((k - 1)
                   + jnp.where(b0 < xv, one, zero)
                   + jnp.where(b1 < xv, one, zero)
                   + jnp.where(b2 < xv, one, zero))
            idx_v[f, pl.ds(v * LANES, LANES)] = idx
        return carry

    def t0_body(ti, carry):
        t0 = wid * T0_PER_W + ti
        pltpu.sync_copy(xT_hbm.at[:, pl.ds(t0 * TPB, TPB)], x_v)
        lax.fori_loop(0, FIELDS, compute_idx, 0)

        gsems = (gsem0, gsem1)
        osems = (osem0, osem1)

        def gather(f, slot):
            pltpu.async_copy(table_hbm.at[idx_v.at[f]],
                             rows_v.at[slot], gsems[slot])

        def wait_gather(slot):
            pltpu.make_async_copy(table_hbm.at[pl.ds(0, TPB)],
                                  rows_v.at[slot], gsems[slot]).wait()

        def wait_owrite(slot):
            pltpu.make_async_copy(out_hbm.at[0, :, pl.ds(0, 1)],
                                  tile_v.at[slot], osems[slot]).wait()

        def transpose(slot):
            rv = rows_v.at[slot]
            tv = tile_v.at[slot]
            for t2 in range(H8):
                for s in range(8):
                    h = jnp.full((LANES,), 8 * t2 + s, jnp.int32)
                    for L in range(TPB // LANES):
                        tv[t2, 0, s, pl.ds(L * LANES, LANES)] = (
                            plsc.load_gather(rv, [bidx[L], h]))

        def owrite(f, slot):
            pltpu.async_copy(tile_v.at[slot],
                             out_hbm.at[f, :, pl.ds(t0, 1)], osems[slot])

        gather(0, 0)

        def f_body(f2, carry):
            f = f2 * 2
            gather(f + 1, 1)
            wait_gather(0)
            transpose(0)

            @pl.when(f2 > 0)
            def _():
                wait_owrite(0)
            owrite(f, 0)

            @pl.when(f2 < FIELDS // 2 - 1)
            def _():
                gather(f + 2, 0)
            wait_gather(1)
            transpose(1)

            @pl.when(f2 > 0)
            def _():
                wait_owrite(1)
            owrite(f + 1, 1)
            return carry

        lax.fori_loop(0, FIELDS // 2, f_body, 0)
        wait_owrite(0)
        wait_owrite(1)
        return carry

    lax.fori_loop(0, T0_PER_W, t0_body, 0)


def kernel(x, table):
    bounds = jnp.linspace(0.0, 1.0, N_BINS - 1, dtype=jnp.float32)
    bounds = jnp.concatenate([bounds, jnp.ones((1,), jnp.float32)])
    out = _emb_kernel(jnp.transpose(x), bounds, table)
    return out.transpose((2, 4, 0, 1, 3)).reshape(BATCH, FIELDS, HIDDEN)


# speedup vs baseline: 6.9951x; 1.8241x over previous
import functools

import jax
import jax.numpy as jnp
from jax import lax
from jax.experimental import pallas as pl
from jax.experimental.pallas import tpu as pltpu
from jax.experimental.pallas import tpu_sc as plsc

N_BINS = 2048
HIDDEN = 64
BATCH = 16384
FIELDS = 100

NC = 2
NS = 16
NW = NC * NS
LANES = 16

TPB = 128
NT0 = BATCH // TPB
T0_PER_W = NT0 // NW
H8 = HIDDEN // 8

_mesh = plsc.VectorSubcoreMesh(core_axis_name="c", subcore_axis_name="s")


@functools.partial(
    pl.kernel,
    mesh=_mesh,
    out_type=jax.ShapeDtypeStruct((FIELDS, H8, NT0, 8, TPB), jnp.float32),
    scratch_types=[
        pltpu.VMEM((N_BINS,), jnp.float32),
        pltpu.VMEM((FIELDS, TPB), jnp.float32),
        pltpu.VMEM((FIELDS, TPB), jnp.int32),
        pltpu.VMEM((2, TPB, HIDDEN), jnp.float32),
        pltpu.VMEM((TPB * (HIDDEN + 1),), jnp.float32),
        pltpu.VMEM((2, H8, 1, 8, TPB), jnp.float32),
        pltpu.SemaphoreType.DMA,
        pltpu.SemaphoreType.DMA,
        pltpu.SemaphoreType.DMA,
        pltpu.SemaphoreType.DMA,
    ],
    compiler_params=pltpu.CompilerParams(
        needs_layout_passes=False, use_tc_tiling_on_sc=False,
        disable_bounds_checks=True),
)
def _emb_kernel(xT_hbm, bounds_hbm, table_hbm, out_hbm,
                bounds_v, x_v, idx_v, rows_v, skew_v, tile_v,
                gsem0, gsem1, osem0, osem1):
    wid = lax.axis_index("s") * NC + lax.axis_index("c")
    pltpu.sync_copy(bounds_hbm, bounds_v)
    iota = lax.iota(jnp.int32, LANES)
    SKW = HIDDEN + 1
    iota_skw = iota * SKW

    def compute_idx(f, carry):
        for v in range(TPB // LANES):
            xv = x_v[f, pl.ds(v * LANES, LANES)]
            k = jnp.clip((xv * jnp.float32(2046.0)).astype(jnp.int32),
                         1, N_BINS - 3)
            b0 = plsc.load_gather(bounds_v, [k - 1])
            b1 = plsc.load_gather(bounds_v, [k])
            b2 = plsc.load_gather(bounds_v, [k + 1])
            one = jnp.int32(1)
            zero = jnp.int32(0)
            idx = ((k - 1)
                   + jnp.where(b0 < xv, one, zero)
                   + jnp.where(b1 < xv, one, zero)
                   + jnp.where(b2 < xv, one, zero))
            idx_v[f, pl.ds(v * LANES, LANES)] = idx
        return carry

    def t0_body(ti, carry):
        t0 = wid * T0_PER_W + ti
        pltpu.sync_copy(xT_hbm.at[:, pl.ds(t0 * TPB, TPB)], x_v)
        lax.fori_loop(0, FIELDS, compute_idx, 0)

        gsems = (gsem0, gsem1)
        osems = (osem0, osem1)

        def gather(f, slot):
            pltpu.async_copy(table_hbm.at[idx_v.at[f]],
                             rows_v.at[slot], gsems[slot])

        def wait_gather(slot):
            pltpu.make_async_copy(table_hbm.at[pl.ds(0, TPB)],
                                  rows_v.at[slot], gsems[slot]).wait()

        def wait_owrite(slot):
            pltpu.make_async_copy(out_hbm.at[0, :, pl.ds(0, 1)],
                                  tile_v.at[slot], osems[slot]).wait()

        def transpose(slot):
            rv = rows_v.at[slot]
            tv = tile_v.at[slot]

            def repack(l4, carry):
                for u in range(4):
                    l = l4 * 4 + u
                    for k in range(HIDDEN // LANES):
                        v = rv[l, pl.ds(k * LANES, LANES)]
                        plsc.store_scatter(
                            skew_v, [iota + (l * SKW + k * LANES)], v)
                return carry

            lax.fori_loop(0, TPB // 4, repack, 0)

            def tbody(h2, carry):
                for u in range(2):
                    h = h2 * 2 + u
                    t2 = h // 8
                    s = h % 8
                    for L in range(TPB // LANES):
                        tv[t2, 0, s, pl.ds(L * LANES, LANES)] = (
                            plsc.load_gather(
                                skew_v, [iota_skw + (L * LANES * SKW + h)]))
                return carry

            lax.fori_loop(0, HIDDEN // 2, tbody, 0)

        def owrite(f, slot):
            pltpu.async_copy(tile_v.at[slot],
                             out_hbm.at[f, :, pl.ds(t0, 1)], osems[slot])

        gather(0, 0)

        def f_body(f2, carry):
            f = f2 * 2
            gather(f + 1, 1)
            wait_gather(0)
            transpose(0)

            @pl.when(f2 > 0)
            def _():
                wait_owrite(0)
            owrite(f, 0)

            @pl.when(f2 < FIELDS // 2 - 1)
            def _():
                gather(f + 2, 0)
            wait_gather(1)
            transpose(1)

            @pl.when(f2 > 0)
            def _():
                wait_owrite(1)
            owrite(f + 1, 1)
            return carry

        lax.fori_loop(0, FIELDS // 2, f_body, 0)
        wait_owrite(0)
        wait_owrite(1)
        return carry

    lax.fori_loop(0, T0_PER_W, t0_body, 0)


def kernel(x, table):
    bounds = jnp.linspace(0.0, 1.0, N_BINS - 1, dtype=jnp.float32)
    bounds = jnp.concatenate([bounds, jnp.ones((1,), jnp.float32)])
    out = _emb_kernel(jnp.transpose(x), bounds, table)
    return out.transpose((2, 4, 0, 1, 3)).reshape(BATCH, FIELDS, HIDDEN)


# speedup vs baseline: 20.9406x; 2.9936x over previous
import functools

import jax
import jax.numpy as jnp
from jax import lax
from jax.experimental import pallas as pl
from jax.experimental.pallas import tpu as pltpu
from jax.experimental.pallas import tpu_sc as plsc

N_BINS = 2048
HIDDEN = 64
BATCH = 16384
FIELDS = 100

NC = 2
NS = 16
NW = NC * NS
LANES = 16

TPB = 128
NT0 = BATCH // TPB
T0_PER_W = NT0 // NW
H8 = HIDDEN // 8

_mesh = plsc.VectorSubcoreMesh(core_axis_name="c", subcore_axis_name="s")


@functools.partial(
    pl.kernel,
    mesh=_mesh,
    out_type=jax.ShapeDtypeStruct((FIELDS, H8, NT0, 8, TPB), jnp.float32),
    scratch_types=[
        pltpu.VMEM((N_BINS,), jnp.float32),
        pltpu.VMEM((FIELDS, TPB), jnp.float32),
        pltpu.VMEM((FIELDS, TPB), jnp.int32),
        pltpu.VMEM((2, TPB, HIDDEN), jnp.float32),
        pltpu.VMEM((TPB * (HIDDEN + 1),), jnp.float32),
        pltpu.VMEM((2, H8, 1, 8, TPB), jnp.float32),
        pltpu.SemaphoreType.DMA,
        pltpu.SemaphoreType.DMA,
        pltpu.SemaphoreType.DMA,
        pltpu.SemaphoreType.DMA,
    ],
    compiler_params=pltpu.CompilerParams(
        needs_layout_passes=False, use_tc_tiling_on_sc=False,
        disable_bounds_checks=True),
)
def _emb_kernel(xT_hbm, bounds_hbm, table_hbm, out_hbm,
                bounds_v, x_v, idx_v, rows_v, skew_v, tile_v,
                gsem0, gsem1, osem0, osem1):
    wid = lax.axis_index("s") * NC + lax.axis_index("c")
    pltpu.sync_copy(bounds_hbm, bounds_v)
    iota = lax.iota(jnp.int32, LANES)
    SKW = HIDDEN + 1
    iota_skw = iota * SKW

    def compute_idx(f):
        for v in range(TPB // LANES):
            xv = x_v[f, pl.ds(v * LANES, LANES)]
            k = jnp.clip((xv * jnp.float32(2046.0)).astype(jnp.int32),
                         1, N_BINS - 3)
            b0 = plsc.load_gather(bounds_v, [k - 1])
            b1 = plsc.load_gather(bounds_v, [k])
            b2 = plsc.load_gather(bounds_v, [k + 1])
            one = jnp.int32(1)
            zero = jnp.int32(0)
            idx = ((k - 1)
                   + jnp.where(b0 < xv, one, zero)
                   + jnp.where(b1 < xv, one, zero)
                   + jnp.where(b2 < xv, one, zero))
            idx_v[f, pl.ds(v * LANES, LANES)] = idx

    def t0_body(ti, carry):
        t0 = wid * T0_PER_W + ti
        pltpu.sync_copy(xT_hbm.at[:, pl.ds(t0 * TPB, TPB)], x_v)
        plsc.parallel_loop(0, FIELDS, 1, unroll=2)(compute_idx)

        gsems = (gsem0, gsem1)
        osems = (osem0, osem1)

        def gather(f, slot):
            pltpu.async_copy(table_hbm.at[idx_v.at[f]],
                             rows_v.at[slot], gsems[slot])

        def wait_gather(slot):
            pltpu.make_async_copy(table_hbm.at[pl.ds(0, TPB)],
                                  rows_v.at[slot], gsems[slot]).wait()

        def wait_owrite(slot):
            pltpu.make_async_copy(out_hbm.at[0, :, pl.ds(0, 1)],
                                  tile_v.at[slot], osems[slot]).wait()

        def transpose(slot):
            rv = rows_v.at[slot]
            tv = tile_v.at[slot]

            @plsc.parallel_loop(0, TPB, 1, unroll=4)
            def repack(l):
                for k in range(HIDDEN // LANES):
                    v = rv[l, pl.ds(k * LANES, LANES)]
                    plsc.store_scatter(
                        skew_v, [iota + (l * SKW + k * LANES)], v)

            @plsc.parallel_loop(0, HIDDEN, 1, unroll=2)
            def tbody(h):
                t2 = h // 8
                s = h % 8
                for L in range(TPB // LANES):
                    tv[t2, 0, s, pl.ds(L * LANES, LANES)] = (
                        plsc.load_gather(
                            skew_v, [iota_skw + (L * LANES * SKW + h)]))

        def owrite(f, slot):
            pltpu.async_copy(tile_v.at[slot],
                             out_hbm.at[f, :, pl.ds(t0, 1)], osems[slot])

        gather(0, 0)

        def f_body(f2, carry):
            f = f2 * 2
            gather(f + 1, 1)
            wait_gather(0)
            transpose(0)

            @pl.when(f2 > 0)
            def _():
                wait_owrite(0)
            owrite(f, 0)

            @pl.when(f2 < FIELDS // 2 - 1)
            def _():
                gather(f + 2, 0)
            wait_gather(1)
            transpose(1)

            @pl.when(f2 > 0)
            def _():
                wait_owrite(1)
            owrite(f + 1, 1)
            return carry

        lax.fori_loop(0, FIELDS // 2, f_body, 0)
        wait_owrite(0)
        wait_owrite(1)
        return carry

    lax.fori_loop(0, T0_PER_W, t0_body, 0)


def kernel(x, table):
    bounds = jnp.linspace(0.0, 1.0, N_BINS - 1, dtype=jnp.float32)
    bounds = jnp.concatenate([bounds, jnp.ones((1,), jnp.float32)])
    out = _emb_kernel(jnp.transpose(x), bounds, table)
    return out.transpose((2, 4, 0, 1, 3)).reshape(BATCH, FIELDS, HIDDEN)
